# Initial kernel scaffold; baseline (speedup 1.0000x reference)
import jax
import jax.numpy as jnp
from jax.experimental import pallas as pl


def kernel(x1, edge_index, edge_weight, W1, b1, W2, b2, gn_weight, gn_bias, gn_mean_scale):
    raise NotImplementedError("write your pallas kernel here")


# speedup vs baseline: 2.6452x; 2.6452x over previous
import jax
import jax.numpy as jnp
from jax import lax
from jax.experimental import pallas as pl
from jax.experimental.pallas import tpu as pltpu
from jax.experimental.pallas import tpu_sc as plsc

N = 10000
E = 320000
D = 128
NC = 2
NS = 16
DF = D // NC
CH = 128
NCHUNK = 160
E_TILE = CH * NCHUNK
E_PAD = NS * E_TILE
ROWS_TILE = 640
N_PAD = NS * ROWS_TILE
ZROWS = 128
VB = 16
BLK = 1000
NBLK = N // BLK


def _k1_body(x1_ref, w1_ref, b1_ref, w2b_ref, b2_ref, x2_ref, y1_ref):
    x = x1_ref[...]
    h = jnp.dot(x, w1_ref[...], preferred_element_type=jnp.float32) + b1_ref[...]
    x2 = jnp.where(h > 0, h, jnp.exp(h) - 1.0)
    x2_ref[0] = x2[:, :DF]
    x2_ref[1] = x2[:, DF:]
    y1_ref[...] = jnp.dot(x, w2b_ref[...], preferred_element_type=jnp.float32) + b2_ref[...]


def _k3_body(p_ref, agg_ref, s_ref, q_ref):
    a = jnp.concatenate([p_ref[0], p_ref[1]], axis=-1).astype(jnp.float32)
    agg_ref[...] = a

    @pl.when(pl.program_id(0) == 0)
    def _init():
        s_ref[...] = jnp.zeros_like(s_ref)
        q_ref[...] = jnp.zeros_like(q_ref)

    s_ref[...] += jnp.sum(a, axis=0, keepdims=True)
    q_ref[...] += jnp.sum(a * a, axis=0, keepdims=True)


def _k4_body(agg_ref, y1_ref, s_ref, q_ref, w2a_ref, gw_ref, gb_ref, gms_ref, o_ref):
    n = jnp.float32(N)
    m = s_ref[...] / n
    t = m * gms_ref[...]
    var = q_ref[...] / n - 2.0 * t * m + t * t
    inv = lax.rsqrt(var + 1e-5)
    d = inv * gw_ref[...]
    cvec = gb_ref[...] - t * d
    corr = jnp.dot(cvec, w2a_ref[...], preferred_element_type=jnp.float32)
    o_ref[...] = (jnp.dot(agg_ref[...] * d, w2a_ref[...],
                          preferred_element_type=jnp.float32)
                  + corr + y1_ref[...])


def _sc_body(x2_hbm, col_hbm, row_hbm, w_hbm, out_hbm,
             colv, rowv, wbuf, rb0, rb1, zb, cb, acc,
             gsem0, gsem1, ssem0, ssem1):
    c = lax.axis_index("c")
    s = lax.axis_index("s")

    pltpu.sync_copy(col_hbm.at[c, s], colv)
    pltpu.sync_copy(row_hbm.at[s], rowv)
    pltpu.sync_copy(w_hbm.at[s], wbuf)

    gdn = lax.GatherDimensionNumbers(
        offset_dims=(), collapsed_slice_dims=(0,), start_index_map=(0,))

    def lane_bcast(v, l):
        idx = jnp.full((VB, 1), l, dtype=jnp.int32)
        return lax.gather(v, idx, gdn, (1,),
                          mode=lax.GatherScatterMode.PROMISE_IN_BOUNDS)

    zeros16 = jnp.zeros((VB,), jnp.float32)

    @pl.loop(0, ZROWS * (DF // VB))
    def _zfill(t):
        zb[t // (DF // VB), pl.ds((t % (DF // VB)) * VB, VB)] = zeros16

    @pl.loop(0, ROWS_TILE // ZROWS)
    def _zcopy(k):
        pltpu.sync_copy(zb, acc.at[pl.ds(s * ROWS_TILE + k * ZROWS, ZROWS)])

    plsc.subcore_barrier()

    def gather(j, rb, sem):
        pltpu.async_copy(x2_hbm.at[colv.at[j]], rb, sem)

    def gather_wait(rb, sem):
        pltpu.make_async_copy(x2_hbm.at[colv.at[0]], rb, sem).wait()

    def scat(j, rb, sem):
        pltpu.async_copy(rb, acc.at[rowv.at[j]], sem, add=True)

    def scat_wait(rb, sem):
        pltpu.make_async_copy(rb, acc.at[rowv.at[0]], sem).wait()

    def scale(rb, j):
        @pl.loop(0, CH // VB)
        def _g(g):
            w16 = wbuf[j, pl.ds(g * VB, VB)]
            for l in range(VB):
                wbc = lane_bcast(w16, l)
                e = g * VB + l
                for q in range(DF // VB):
                    sl = pl.ds(q * VB, VB)
                    rb[e, sl] = rb[e, sl] * wbc

    gather(0, rb0, gsem0)

    @pl.loop(0, NCHUNK // 2)
    def _main(k):
        j = k * 2
        gather_wait(rb0, gsem0)
        scale(rb0, j)

        @pl.when(k > 0)
        def _():
            scat_wait(rb1, ssem1)

        gather(j + 1, rb1, gsem1)
        scat(j, rb0, ssem0)

        gather_wait(rb1, gsem1)
        scale(rb1, j + 1)
        scat_wait(rb0, ssem0)

        @pl.when(k < NCHUNK // 2 - 1)
        def _():
            gather(j + 2, rb0, gsem0)

        scat(j + 1, rb1, ssem1)

    scat_wait(rb1, ssem1)

    plsc.subcore_barrier()

    iota16 = lax.iota(jnp.int32, VB)

    @pl.loop(0, ROWS_TILE // ZROWS)
    def _dump(k):
        r0 = s * ROWS_TILE + k * ZROWS
        pltpu.sync_copy(acc.at[pl.ds(r0, ZROWS)], zb)

        @pl.loop(0, ZROWS)
        def _cvt(r):
            ridx = jnp.full((VB,), r, dtype=jnp.int32)
            for g in range(DF // (2 * VB)):
                ca = g * 2 * VB + 2 * iota16
                a = plsc.load_gather(zb, [ridx, ca])
                b = plsc.load_gather(zb, [ridx, ca + 1])
                cb[r, pl.ds(g * 2 * VB, 2 * VB)] = plsc.pack(
                    a, b, format=plsc.PackFormat.INTERLEAVED)

        pltpu.sync_copy(cb, out_hbm.at[c, pl.ds(r0, ZROWS)])


def _sc_aggregate(x2flat, col4, row3, w3):
    mesh = plsc.VectorSubcoreMesh(core_axis_name="c", subcore_axis_name="s",
                                  num_cores=NC, num_subcores=NS)
    return pl.kernel(
        _sc_body,
        out_type=jax.ShapeDtypeStruct((NC, N_PAD, DF), jnp.bfloat16),
        mesh=mesh,
        scratch_types=[
            pltpu.VMEM((NCHUNK, CH), jnp.int32),
            pltpu.VMEM((NCHUNK, CH), jnp.int32),
            pltpu.VMEM((NCHUNK, CH), jnp.float32),
            pltpu.VMEM((CH, DF), jnp.float32),
            pltpu.VMEM((CH, DF), jnp.float32),
            pltpu.VMEM((ZROWS, DF), jnp.float32),
            pltpu.VMEM((ZROWS, DF), jnp.bfloat16),
            pltpu.VMEM_SHARED((N_PAD, DF), jnp.float32),
            pltpu.SemaphoreType.DMA,
            pltpu.SemaphoreType.DMA,
            pltpu.SemaphoreType.DMA,
            pltpu.SemaphoreType.DMA,
        ],
        compiler_params=pltpu.CompilerParams(use_tc_tiling_on_sc=False,
                                             needs_layout_passes=False),
    )(x2flat, col4, row3, w3)


def kernel(x1, edge_index, edge_weight, W1, b1, W2, b2,
           gn_weight, gn_bias, gn_mean_scale):
    pad = E_PAD - E
    colp = jnp.pad(edge_index[1], (0, pad))
    col4 = jnp.stack([colp, colp + N]).reshape(NC, NS, NCHUNK, CH)
    row3 = jnp.pad(edge_index[0], (0, pad)).reshape(NS, NCHUNK, CH)
    w3 = jnp.pad(edge_weight, (0, pad)).reshape(NS, NCHUNK, CH)
    W2a = W2[:D]
    W2b = W2[D:]
    b1r = b1.reshape(1, D)
    b2r = b2.reshape(1, D)

    x2s, y1 = pl.pallas_call(
        _k1_body,
        grid=(NBLK,),
        in_specs=[
            pl.BlockSpec((BLK, D), lambda i: (i, 0)),
            pl.BlockSpec((D, D), lambda i: (0, 0)),
            pl.BlockSpec((1, D), lambda i: (0, 0)),
            pl.BlockSpec((D, D), lambda i: (0, 0)),
            pl.BlockSpec((1, D), lambda i: (0, 0)),
        ],
        out_specs=[
            pl.BlockSpec((NC, BLK, DF), lambda i: (0, i, 0)),
            pl.BlockSpec((BLK, D), lambda i: (i, 0)),
        ],
        out_shape=[
            jax.ShapeDtypeStruct((NC, N, DF), jnp.float32),
            jax.ShapeDtypeStruct((N, D), jnp.float32),
        ],
    )(x1, W1, b1r, W2b, b2r)

    partials = _sc_aggregate(x2s.reshape(NC * N, DF), col4, row3, w3)

    agg, S, Q = pl.pallas_call(
        _k3_body,
        grid=(NBLK,),
        in_specs=[pl.BlockSpec((NC, BLK, DF), lambda i: (0, i, 0))],
        out_specs=[
            pl.BlockSpec((BLK, D), lambda i: (i, 0)),
            pl.BlockSpec((1, D), lambda i: (0, 0)),
            pl.BlockSpec((1, D), lambda i: (0, 0)),
        ],
        out_shape=[
            jax.ShapeDtypeStruct((N, D), jnp.float32),
            jax.ShapeDtypeStruct((1, D), jnp.float32),
            jax.ShapeDtypeStruct((1, D), jnp.float32),
        ],
    )(partials)

    out = pl.pallas_call(
        _k4_body,
        grid=(NBLK,),
        in_specs=[
            pl.BlockSpec((BLK, D), lambda i: (i, 0)),
            pl.BlockSpec((BLK, D), lambda i: (i, 0)),
            pl.BlockSpec((1, D), lambda i: (0, 0)),
            pl.BlockSpec((1, D), lambda i: (0, 0)),
            pl.BlockSpec((D, D), lambda i: (0, 0)),
            pl.BlockSpec((1, D), lambda i: (0, 0)),
            pl.BlockSpec((1, D), lambda i: (0, 0)),
            pl.BlockSpec((1, D), lambda i: (0, 0)),
        ],
        out_specs=pl.BlockSpec((BLK, D), lambda i: (i, 0)),
        out_shape=jax.ShapeDtypeStruct((N, D), jnp.float32),
    )(agg, y1, S, Q, W2a,
      gn_weight.reshape(1, D), gn_bias.reshape(1, D),
      gn_mean_scale.reshape(1, D))

    return out


# speedup vs baseline: 3.5971x; 1.3599x over previous
import jax
import jax.numpy as jnp
from jax import lax
from jax.experimental import pallas as pl
from jax.experimental.pallas import tpu as pltpu
from jax.experimental.pallas import tpu_sc as plsc

N = 10000
E = 320000
D = 128
NC = 2
NS = 16
DF = D // NC
CH = 128
NCHUNK = 160
E_TILE = CH * NCHUNK
E_PAD = NS * E_TILE
ROWS_TILE = 640
N_PAD = NS * ROWS_TILE
ZROWS = 128
VB = 16
BLK = 1000
NBLK = N // BLK


def _k1_body(x1_ref, w1_ref, b1_ref, w2b_ref, b2_ref, x2_ref, y1_ref):
    x = x1_ref[...]
    h = jnp.dot(x, w1_ref[...], preferred_element_type=jnp.float32) + b1_ref[...]
    x2 = jnp.where(h > 0, h, jnp.exp(h) - 1.0)
    x2_ref[0] = x2[:, :DF]
    x2_ref[1] = x2[:, DF:]
    y1_ref[...] = jnp.dot(x, w2b_ref[...], preferred_element_type=jnp.float32) + b2_ref[...]


def _k3_body(p_ref, agg_ref, s_ref, q_ref):
    a = jnp.concatenate([p_ref[0], p_ref[1]], axis=-1).astype(jnp.float32)
    agg_ref[...] = a

    @pl.when(pl.program_id(0) == 0)
    def _init():
        s_ref[...] = jnp.zeros_like(s_ref)
        q_ref[...] = jnp.zeros_like(q_ref)

    s_ref[...] += jnp.sum(a, axis=0, keepdims=True)
    q_ref[...] += jnp.sum(a * a, axis=0, keepdims=True)


def _k4_body(agg_ref, y1_ref, s_ref, q_ref, w2a_ref, gw_ref, gb_ref, gms_ref, o_ref):
    n = jnp.float32(N)
    m = s_ref[...] / n
    t = m * gms_ref[...]
    var = q_ref[...] / n - 2.0 * t * m + t * t
    inv = lax.rsqrt(var + 1e-5)
    d = inv * gw_ref[...]
    cvec = gb_ref[...] - t * d
    corr = jnp.dot(cvec, w2a_ref[...], preferred_element_type=jnp.float32)
    o_ref[...] = (jnp.dot(agg_ref[...] * d, w2a_ref[...],
                          preferred_element_type=jnp.float32)
                  + corr + y1_ref[...])


def _sc_body(x2_hbm, cr_hbm, w_hbm, out_hbm,
             crv, wbuf, colb, rowb, rb0, rb1, rb2, rb3, zb, cb, acc,
             gsem0, gsem1, gsem2, gsem3, ssem0, ssem1, ssem2, ssem3):
    c = lax.axis_index("c")
    s = lax.axis_index("s")

    pltpu.sync_copy(cr_hbm.at[c, s], crv)
    pltpu.sync_copy(w_hbm.at[s], wbuf)

    gdn = lax.GatherDimensionNumbers(
        offset_dims=(), collapsed_slice_dims=(0,), start_index_map=(0,))

    def lane_bcast(v, l):
        idx = jnp.full((VB, 1), l, dtype=jnp.int32)
        return lax.gather(v, idx, gdn, (1,),
                          mode=lax.GatherScatterMode.PROMISE_IN_BOUNDS)

    zeros16 = jnp.zeros((VB,), jnp.float32)

    @pl.loop(0, ZROWS * (DF // VB))
    def _zfill(t):
        zb[t // (DF // VB), pl.ds((t % (DF // VB)) * VB, VB)] = zeros16

    @pl.loop(0, ROWS_TILE // ZROWS)
    def _zcopy(k):
        pltpu.sync_copy(zb, acc.at[pl.ds(s * ROWS_TILE + k * ZROWS, ZROWS)])

    plsc.subcore_barrier()

    def unpack_idx(m, t):
        @pl.loop(0, CH // VB)
        def _u(g):
            v = crv[m, pl.ds(g * VB, VB)]
            colb[t, pl.ds(g * VB, VB)] = jnp.bitwise_and(v, 32767)
            rowb[t, pl.ds(g * VB, VB)] = jnp.right_shift(v, 15)

    def gather(t, rb, sem):
        pltpu.async_copy(x2_hbm.at[colb.at[t]], rb, sem)

    def gather_wait(rb, sem):
        pltpu.make_async_copy(x2_hbm.at[colb.at[0]], rb, sem).wait()

    def scat(t, rb, sem):
        pltpu.async_copy(rb, acc.at[rowb.at[t]], sem, add=True)

    def scat_wait(rb, sem):
        pltpu.make_async_copy(rb, acc.at[rowb.at[0]], sem).wait()

    def scale(rb, j):
        @pl.loop(0, CH // VB)
        def _g(g):
            w16 = wbuf[j, pl.ds(g * VB, VB)]
            for l in range(VB):
                wbc = lane_bcast(w16, l)
                e = g * VB + l
                for q in range(DF // VB):
                    sl = pl.ds(q * VB, VB)
                    rb[e, sl] = rb[e, sl] * wbc

    rbs = [rb0, rb1, rb2, rb3]
    gsems = [gsem0, gsem1, gsem2, gsem3]
    ssems = [ssem0, ssem1, ssem2, ssem3]

    unpack_idx(0, 0)
    gather(0, rb0, gsem0)
    unpack_idx(1, 1)
    gather(1, rb1, gsem1)
    unpack_idx(2, 2)
    gather(2, rb2, gsem2)

    @pl.loop(0, NCHUNK // 4)
    def _main(k):
        j = k * 4
        for t in range(4):
            jt = j + t
            tn = (t + 3) % 4

            if t == 0:
                @pl.when(k > 0)
                def _():
                    scat_wait(rbs[tn], ssems[tn])
                unpack_idx(jt + 3, tn)
                gather(tn, rbs[tn], gsems[tn])
            else:
                scat_wait(rbs[tn], ssems[tn])

                @pl.when(k < NCHUNK // 4 - 1)
                def _():
                    unpack_idx(jt + 3, tn)
                    gather(tn, rbs[tn], gsems[tn])

            gather_wait(rbs[t], gsems[t])
            scale(rbs[t], jt)
            scat(t, rbs[t], ssems[t])

    scat_wait(rb3, ssem3)

    plsc.subcore_barrier()

    iota16 = lax.iota(jnp.int32, VB)

    @pl.loop(0, ROWS_TILE // ZROWS)
    def _dump(k):
        r0 = s * ROWS_TILE + k * ZROWS
        pltpu.sync_copy(acc.at[pl.ds(r0, ZROWS)], zb)

        @pl.loop(0, ZROWS)
        def _cvt(r):
            ridx = jnp.full((VB,), r, dtype=jnp.int32)
            for g in range(DF // (2 * VB)):
                ca = g * 2 * VB + 2 * iota16
                a = plsc.load_gather(zb, [ridx, ca])
                b = plsc.load_gather(zb, [ridx, ca + 1])
                cb[r, pl.ds(g * 2 * VB, 2 * VB)] = plsc.pack(
                    a, b, format=plsc.PackFormat.INTERLEAVED)

        pltpu.sync_copy(cb, out_hbm.at[c, pl.ds(r0, ZROWS)])


def _sc_aggregate(x2flat, cr4, w3):
    mesh = plsc.VectorSubcoreMesh(core_axis_name="c", subcore_axis_name="s",
                                  num_cores=NC, num_subcores=NS)
    return pl.kernel(
        _sc_body,
        out_type=jax.ShapeDtypeStruct((NC, N_PAD, DF), jnp.bfloat16),
        mesh=mesh,
        scratch_types=[
            pltpu.VMEM((NCHUNK, CH), jnp.int32),
            pltpu.VMEM((NCHUNK, CH), jnp.float32),
            pltpu.VMEM((4, CH), jnp.int32),
            pltpu.VMEM((4, CH), jnp.int32),
            pltpu.VMEM((CH, DF), jnp.float32),
            pltpu.VMEM((CH, DF), jnp.float32),
            pltpu.VMEM((CH, DF), jnp.float32),
            pltpu.VMEM((CH, DF), jnp.float32),
            pltpu.VMEM((ZROWS, DF), jnp.float32),
            pltpu.VMEM((ZROWS, DF), jnp.bfloat16),
            pltpu.VMEM_SHARED((N_PAD, DF), jnp.float32),
            pltpu.SemaphoreType.DMA,
            pltpu.SemaphoreType.DMA,
            pltpu.SemaphoreType.DMA,
            pltpu.SemaphoreType.DMA,
            pltpu.SemaphoreType.DMA,
            pltpu.SemaphoreType.DMA,
            pltpu.SemaphoreType.DMA,
            pltpu.SemaphoreType.DMA,
        ],
        compiler_params=pltpu.CompilerParams(use_tc_tiling_on_sc=False,
                                             needs_layout_passes=False),
    )(x2flat, cr4, w3)


def kernel(x1, edge_index, edge_weight, W1, b1, W2, b2,
           gn_weight, gn_bias, gn_mean_scale):
    pad = E_PAD - E
    packed = jnp.pad(edge_index[0] * 32768 + edge_index[1], (0, pad))
    cr4 = jnp.stack([packed, packed + N]).reshape(NC, NS, NCHUNK, CH)
    w3 = jnp.pad(edge_weight, (0, pad)).reshape(NS, NCHUNK, CH)
    W2a = W2[:D]
    W2b = W2[D:]
    b1r = b1.reshape(1, D)
    b2r = b2.reshape(1, D)

    x2s, y1 = pl.pallas_call(
        _k1_body,
        grid=(NBLK,),
        in_specs=[
            pl.BlockSpec((BLK, D), lambda i: (i, 0)),
            pl.BlockSpec((D, D), lambda i: (0, 0)),
            pl.BlockSpec((1, D), lambda i: (0, 0)),
            pl.BlockSpec((D, D), lambda i: (0, 0)),
            pl.BlockSpec((1, D), lambda i: (0, 0)),
        ],
        out_specs=[
            pl.BlockSpec((NC, BLK, DF), lambda i: (0, i, 0)),
            pl.BlockSpec((BLK, D), lambda i: (i, 0)),
        ],
        out_shape=[
            jax.ShapeDtypeStruct((NC, N, DF), jnp.float32),
            jax.ShapeDtypeStruct((N, D), jnp.float32),
        ],
    )(x1, W1, b1r, W2b, b2r)

    partials = _sc_aggregate(x2s.reshape(NC * N, DF), cr4, w3)

    agg, S, Q = pl.pallas_call(
        _k3_body,
        grid=(NBLK,),
        in_specs=[pl.BlockSpec((NC, BLK, DF), lambda i: (0, i, 0))],
        out_specs=[
            pl.BlockSpec((BLK, D), lambda i: (i, 0)),
            pl.BlockSpec((1, D), lambda i: (0, 0)),
            pl.BlockSpec((1, D), lambda i: (0, 0)),
        ],
        out_shape=[
            jax.ShapeDtypeStruct((N, D), jnp.float32),
            jax.ShapeDtypeStruct((1, D), jnp.float32),
            jax.ShapeDtypeStruct((1, D), jnp.float32),
        ],
    )(partials)

    out = pl.pallas_call(
        _k4_body,
        grid=(NBLK,),
        in_specs=[
            pl.BlockSpec((BLK, D), lambda i: (i, 0)),
            pl.BlockSpec((BLK, D), lambda i: (i, 0)),
            pl.BlockSpec((1, D), lambda i: (0, 0)),
            pl.BlockSpec((1, D), lambda i: (0, 0)),
            pl.BlockSpec((D, D), lambda i: (0, 0)),
            pl.BlockSpec((1, D), lambda i: (0, 0)),
            pl.BlockSpec((1, D), lambda i: (0, 0)),
            pl.BlockSpec((1, D), lambda i: (0, 0)),
        ],
        out_specs=pl.BlockSpec((BLK, D), lambda i: (i, 0)),
        out_shape=jax.ShapeDtypeStruct((N, D), jnp.float32),
    )(agg, y1, S, Q, W2a,
      gn_weight.reshape(1, D), gn_bias.reshape(1, D),
      gn_mean_scale.reshape(1, D))

    return out


# speedup vs baseline: 5.0852x; 1.4137x over previous
import jax
import jax.numpy as jnp
from jax import lax
from jax.experimental import pallas as pl
from jax.experimental.pallas import tpu as pltpu
from jax.experimental.pallas import tpu_sc as plsc

N = 10000
E = 320000
D = 128
NC = 2
NS = 16
DF = D // NC
CH = 128
NCHUNK = 160
E_TILE = CH * NCHUNK
E_PAD = NS * E_TILE
ROWS_TILE = 640
N_PAD = NS * ROWS_TILE
ZROWS = 128
VB = 16
BLK = 1000
NBLK = N // BLK


def _k1_body(x1_ref, w1_ref, b1_ref, w2b_ref, b2_ref, x2_ref, y1_ref):
    x = x1_ref[...]
    h = jnp.dot(x, w1_ref[...], preferred_element_type=jnp.float32) + b1_ref[...]
    x2 = jnp.where(h > 0, h, jnp.exp(h) - 1.0)
    x2_ref[0] = x2[:, :DF]
    x2_ref[1] = x2[:, DF:]
    y1_ref[...] = jnp.dot(x, w2b_ref[...], preferred_element_type=jnp.float32) + b2_ref[...]


def _k3_body(p_ref, agg_ref, s_ref, q_ref):
    a = jnp.concatenate([p_ref[0], p_ref[1]], axis=-1).astype(jnp.float32)
    agg_ref[...] = a

    @pl.when(pl.program_id(0) == 0)
    def _init():
        s_ref[...] = jnp.zeros_like(s_ref)
        q_ref[...] = jnp.zeros_like(q_ref)

    s_ref[...] += jnp.sum(a, axis=0, keepdims=True)
    q_ref[...] += jnp.sum(a * a, axis=0, keepdims=True)


def _k4_body(agg_ref, y1_ref, s_ref, q_ref, w2a_ref, gw_ref, gb_ref, gms_ref, o_ref):
    n = jnp.float32(N)
    m = s_ref[...] / n
    t = m * gms_ref[...]
    var = q_ref[...] / n - 2.0 * t * m + t * t
    inv = lax.rsqrt(var + 1e-5)
    d = inv * gw_ref[...]
    cvec = gb_ref[...] - t * d
    corr = jnp.dot(cvec, w2a_ref[...], preferred_element_type=jnp.float32)
    o_ref[...] = (jnp.dot(agg_ref[...] * d, w2a_ref[...],
                          preferred_element_type=jnp.float32)
                  + corr + y1_ref[...])


def _sc_body(x2_hbm, cr_hbm, w_hbm, out_hbm,
             crv, wbuf, colb, rowb, rb0, rb1, rb2, rb3, zb, cb, acc,
             gsem0, gsem1, gsem2, gsem3, ssem0, ssem1, ssem2, ssem3):
    c = lax.axis_index("c")
    s = lax.axis_index("s")

    pltpu.sync_copy(cr_hbm.at[c, s], crv)
    pltpu.sync_copy(w_hbm.at[s], wbuf)

    gdn = lax.GatherDimensionNumbers(
        offset_dims=(), collapsed_slice_dims=(0,), start_index_map=(0,))

    def lane_bcast(v, l):
        idx = jnp.full((VB, 1), l, dtype=jnp.int32)
        return lax.gather(v, idx, gdn, (1,),
                          mode=lax.GatherScatterMode.PROMISE_IN_BOUNDS)

    zeros16 = jnp.zeros((VB,), jnp.float32)

    @pl.loop(0, ZROWS * (DF // VB))
    def _zfill(t):
        zb[t // (DF // VB), pl.ds((t % (DF // VB)) * VB, VB)] = zeros16

    @pl.loop(0, ROWS_TILE // ZROWS)
    def _zcopy(k):
        pltpu.sync_copy(zb, acc.at[pl.ds(s * ROWS_TILE + k * ZROWS, ZROWS)])

    plsc.subcore_barrier()

    def unpack_idx(m, t):
        @pl.loop(0, CH // VB)
        def _u(g):
            v = crv[m, pl.ds(g * VB, VB)]
            colb[t, pl.ds(g * VB, VB)] = jnp.bitwise_and(v, 32767)
            rowb[t, pl.ds(g * VB, VB)] = jnp.right_shift(v, 15)

    def gather(t, rb, sem):
        pltpu.async_copy(x2_hbm.at[colb.at[t]], rb, sem)

    def gather_wait(rb, sem):
        pltpu.make_async_copy(x2_hbm.at[colb.at[0]], rb, sem).wait()

    def scat(t, rb, sem):
        pltpu.async_copy(rb, acc.at[rowb.at[t]], sem, add=True)

    def scat_wait(rb, sem):
        pltpu.make_async_copy(rb, acc.at[rowb.at[0]], sem).wait()

    def scale(rb, j):
        @plsc.parallel_loop(0, CH // VB, unroll=2)
        def _g(g):
            w16 = wbuf[j, pl.ds(g * VB, VB)]
            for l in range(VB):
                wbc = lane_bcast(w16, l)
                e = g * VB + l
                for q in range(DF // VB):
                    sl = pl.ds(q * VB, VB)
                    rb[e, sl] = rb[e, sl] * wbc

    rbs = [rb0, rb1, rb2, rb3]
    gsems = [gsem0, gsem1, gsem2, gsem3]
    ssems = [ssem0, ssem1, ssem2, ssem3]

    unpack_idx(0, 0)
    gather(0, rb0, gsem0)
    unpack_idx(1, 1)
    gather(1, rb1, gsem1)
    unpack_idx(2, 2)
    gather(2, rb2, gsem2)

    @pl.loop(0, NCHUNK // 4)
    def _main(k):
        j = k * 4
        for t in range(4):
            jt = j + t
            tn = (t + 3) % 4

            if t == 0:
                @pl.when(k > 0)
                def _():
                    scat_wait(rbs[tn], ssems[tn])
                unpack_idx(jt + 3, tn)
                gather(tn, rbs[tn], gsems[tn])
            else:
                scat_wait(rbs[tn], ssems[tn])

                @pl.when(k < NCHUNK // 4 - 1)
                def _():
                    unpack_idx(jt + 3, tn)
                    gather(tn, rbs[tn], gsems[tn])

            gather_wait(rbs[t], gsems[t])
            scale(rbs[t], jt)
            scat(t, rbs[t], ssems[t])

    scat_wait(rb3, ssem3)

    plsc.subcore_barrier()

    iota16 = lax.iota(jnp.int32, VB)

    @pl.loop(0, ROWS_TILE // ZROWS)
    def _dump(k):
        r0 = s * ROWS_TILE + k * ZROWS
        pltpu.sync_copy(acc.at[pl.ds(r0, ZROWS)], zb)

        @pl.loop(0, ZROWS)
        def _cvt(r):
            ridx = jnp.full((VB,), r, dtype=jnp.int32)
            for g in range(DF // (2 * VB)):
                ca = g * 2 * VB + 2 * iota16
                a = plsc.load_gather(zb, [ridx, ca])
                b = plsc.load_gather(zb, [ridx, ca + 1])
                cb[r, pl.ds(g * 2 * VB, 2 * VB)] = plsc.pack(
                    a, b, format=plsc.PackFormat.INTERLEAVED)

        pltpu.sync_copy(cb, out_hbm.at[c, pl.ds(r0, ZROWS)])


def _sc_aggregate(x2flat, cr4, w3):
    mesh = plsc.VectorSubcoreMesh(core_axis_name="c", subcore_axis_name="s",
                                  num_cores=NC, num_subcores=NS)
    return pl.kernel(
        _sc_body,
        out_type=jax.ShapeDtypeStruct((NC, N_PAD, DF), jnp.bfloat16),
        mesh=mesh,
        scratch_types=[
            pltpu.VMEM((NCHUNK, CH), jnp.int32),
            pltpu.VMEM((NCHUNK, CH), jnp.float32),
            pltpu.VMEM((4, CH), jnp.int32),
            pltpu.VMEM((4, CH), jnp.int32),
            pltpu.VMEM((CH, DF), jnp.float32),
            pltpu.VMEM((CH, DF), jnp.float32),
            pltpu.VMEM((CH, DF), jnp.float32),
            pltpu.VMEM((CH, DF), jnp.float32),
            pltpu.VMEM((ZROWS, DF), jnp.float32),
            pltpu.VMEM((ZROWS, DF), jnp.bfloat16),
            pltpu.VMEM_SHARED((N_PAD, DF), jnp.float32),
            pltpu.SemaphoreType.DMA,
            pltpu.SemaphoreType.DMA,
            pltpu.SemaphoreType.DMA,
            pltpu.SemaphoreType.DMA,
            pltpu.SemaphoreType.DMA,
            pltpu.SemaphoreType.DMA,
            pltpu.SemaphoreType.DMA,
            pltpu.SemaphoreType.DMA,
        ],
        compiler_params=pltpu.CompilerParams(use_tc_tiling_on_sc=False,
                                             needs_layout_passes=False),
    )(x2flat, cr4, w3)


def kernel(x1, edge_index, edge_weight, W1, b1, W2, b2,
           gn_weight, gn_bias, gn_mean_scale):
    pad = E_PAD - E
    packed = jnp.pad(edge_index[0] * 32768 + edge_index[1], (0, pad))
    cr4 = jnp.stack([packed, packed + N]).reshape(NC, NS, NCHUNK, CH)
    w3 = jnp.pad(edge_weight, (0, pad)).reshape(NS, NCHUNK, CH)
    W2a = W2[:D]
    W2b = W2[D:]
    b1r = b1.reshape(1, D)
    b2r = b2.reshape(1, D)

    x2s, y1 = pl.pallas_call(
        _k1_body,
        grid=(NBLK,),
        in_specs=[
            pl.BlockSpec((BLK, D), lambda i: (i, 0)),
            pl.BlockSpec((D, D), lambda i: (0, 0)),
            pl.BlockSpec((1, D), lambda i: (0, 0)),
            pl.BlockSpec((D, D), lambda i: (0, 0)),
            pl.BlockSpec((1, D), lambda i: (0, 0)),
        ],
        out_specs=[
            pl.BlockSpec((NC, BLK, DF), lambda i: (0, i, 0)),
            pl.BlockSpec((BLK, D), lambda i: (i, 0)),
        ],
        out_shape=[
            jax.ShapeDtypeStruct((NC, N, DF), jnp.float32),
            jax.ShapeDtypeStruct((N, D), jnp.float32),
        ],
    )(x1, W1, b1r, W2b, b2r)

    partials = _sc_aggregate(x2s.reshape(NC * N, DF), cr4, w3)

    agg, S, Q = pl.pallas_call(
        _k3_body,
        grid=(NBLK,),
        in_specs=[pl.BlockSpec((NC, BLK, DF), lambda i: (0, i, 0))],
        out_specs=[
            pl.BlockSpec((BLK, D), lambda i: (i, 0)),
            pl.BlockSpec((1, D), lambda i: (0, 0)),
            pl.BlockSpec((1, D), lambda i: (0, 0)),
        ],
        out_shape=[
            jax.ShapeDtypeStruct((N, D), jnp.float32),
            jax.ShapeDtypeStruct((1, D), jnp.float32),
            jax.ShapeDtypeStruct((1, D), jnp.float32),
        ],
    )(partials)

    out = pl.pallas_call(
        _k4_body,
        grid=(NBLK,),
        in_specs=[
            pl.BlockSpec((BLK, D), lambda i: (i, 0)),
            pl.BlockSpec((BLK, D), lambda i: (i, 0)),
            pl.BlockSpec((1, D), lambda i: (0, 0)),
            pl.BlockSpec((1, D), lambda i: (0, 0)),
            pl.BlockSpec((D, D), lambda i: (0, 0)),
            pl.BlockSpec((1, D), lambda i: (0, 0)),
            pl.BlockSpec((1, D), lambda i: (0, 0)),
            pl.BlockSpec((1, D), lambda i: (0, 0)),
        ],
        out_specs=pl.BlockSpec((BLK, D), lambda i: (i, 0)),
        out_shape=jax.ShapeDtypeStruct((N, D), jnp.float32),
    )(agg, y1, S, Q, W2a,
      gn_weight.reshape(1, D), gn_bias.reshape(1, D),
      gn_mean_scale.reshape(1, D))

    return out
